# baseline (speedup 1.0000x reference)
import functools

import jax
import jax.numpy as jnp
from jax.experimental import pallas as pl
from jax.experimental.pallas import tpu as pltpu

HIDDEN = 1024
FFN = 2048
NUM_EXPERTS = 16
TOP_K = 2
CHUNK = 1024
N_CHUNKS = FFN // CHUNK
NSTEPS = NUM_EXPERTS * N_CHUNKS
NBUF = 4


def _moe_kernel(x_ref, gw_ref, wgu_hbm, wd_hbm, out_ref, logits_ref,
                wg_buf, wu_buf, wd_buf, combine_ref,
                wg_sem, wu_sem, wd_sem):

    def copies(s, p):
        e = s // N_CHUNKS
        c = s % N_CHUNKS
        return (
            pltpu.make_async_copy(
                wgu_hbm.at[e, :, pl.ds(c * CHUNK, CHUNK)],
                wg_buf.at[p], wg_sem.at[p]),
            pltpu.make_async_copy(
                wgu_hbm.at[e, :, pl.ds(FFN + c * CHUNK, CHUNK)],
                wu_buf.at[p], wu_sem.at[p]),
            pltpu.make_async_copy(
                wd_hbm.at[e, pl.ds(c * CHUNK, CHUNK), :],
                wd_buf.at[p], wd_sem.at[p]),
        )

    def start(s, p):
        for cp in copies(s, p):
            cp.start()

    def wait(s, p):
        for cp in copies(s, p):
            cp.wait()

    x = x_ref[...]
    logits = jax.lax.dot_general(
        x, gw_ref[...], dimension_numbers=(((1,), (1,)), ((), ())),
        preferred_element_type=jnp.float32)
    logits_ref[...] = logits
    probs = jax.nn.softmax(logits, axis=-1)
    eidx = jax.lax.broadcasted_iota(jnp.int32, probs.shape, 1)
    p1 = jnp.max(probs, axis=-1, keepdims=True)
    i1 = jnp.min(jnp.where(probs >= p1, eidx, NUM_EXPERTS), axis=-1,
                 keepdims=True)
    sel1 = eidx == i1
    probs2 = jnp.where(sel1, -jnp.inf, probs)
    p2 = jnp.max(probs2, axis=-1, keepdims=True)
    i2 = jnp.min(jnp.where(probs2 >= p2, eidx, NUM_EXPERTS), axis=-1,
                 keepdims=True)
    sel2 = eidx == i2
    combine_ref[...] = (jnp.where(sel1, p1, 0.0)
                        + jnp.where(sel2, p2, 0.0)) / (p1 + p2)

    out_ref[...] = jnp.zeros(out_ref.shape, out_ref.dtype)

    for s in range(NBUF):
        start(s, s)

    def group(g, carry):
        for p in range(NBUF):
            s = g * NBUF + p
            wait(s, p)
            gate = jnp.dot(x, wg_buf[p], preferred_element_type=jnp.float32)
            up = jnp.dot(x, wu_buf[p], preferred_element_type=jnp.float32)
            hidden = gate * jax.nn.sigmoid(gate) * up
            down = jnp.dot(hidden, wd_buf[p],
                           preferred_element_type=jnp.float32)
            combine = combine_ref[...]
            lane = jax.lax.broadcasted_iota(jnp.int32, combine.shape, 1)
            e = s // N_CHUNKS
            col = jnp.sum(jnp.where(lane == e, combine, 0.0), axis=-1,
                          keepdims=True)
            out_ref[...] = out_ref[...] + col * down

            @pl.when(s + NBUF < NSTEPS)
            def _():
                start(s + NBUF, p)
        return carry

    jax.lax.fori_loop(0, NSTEPS // NBUF, group, 0)


@functools.partial(jax.jit, static_argnames=())
def kernel(hidden_states, gate_w, w_gate_up, w_down):
    b, s, d = hidden_states.shape
    t = b * s
    x = hidden_states.reshape(t, d)

    out, logits = pl.pallas_call(
        _moe_kernel,
        in_specs=[
            pl.BlockSpec((t, d), lambda: (0, 0)),
            pl.BlockSpec((NUM_EXPERTS, d), lambda: (0, 0)),
            pl.BlockSpec(memory_space=pl.ANY),
            pl.BlockSpec(memory_space=pl.ANY),
        ],
        out_specs=[
            pl.BlockSpec((t, d), lambda: (0, 0)),
            pl.BlockSpec((t, NUM_EXPERTS), lambda: (0, 0)),
        ],
        out_shape=[
            jax.ShapeDtypeStruct((t, d), jnp.float32),
            jax.ShapeDtypeStruct((t, NUM_EXPERTS), jnp.float32),
        ],
        scratch_shapes=[
            pltpu.VMEM((NBUF, HIDDEN, CHUNK), jnp.float32),
            pltpu.VMEM((NBUF, HIDDEN, CHUNK), jnp.float32),
            pltpu.VMEM((NBUF, CHUNK, HIDDEN), jnp.float32),
            pltpu.VMEM((t, NUM_EXPERTS), jnp.float32),
            pltpu.SemaphoreType.DMA((NBUF,)),
            pltpu.SemaphoreType.DMA((NBUF,)),
            pltpu.SemaphoreType.DMA((NBUF,)),
        ],
    )(x, gate_w, w_gate_up, w_down)

    return out.reshape(b, s, d), logits


# speedup vs baseline: 1.0104x; 1.0104x over previous
import functools

import jax
import jax.numpy as jnp
from jax.experimental import pallas as pl
from jax.experimental.pallas import tpu as pltpu

HIDDEN = 1024
FFN = 2048
NUM_EXPERTS = 16
TOP_K = 2
CHUNK = 1024
N_CHUNKS = FFN // CHUNK
NSTEPS = NUM_EXPERTS * N_CHUNKS
NBUF = 4


def _moe_kernel(x_ref, gw_ref, wgu_hbm, wd_hbm, out_ref, logits_ref,
                wg_buf, wu_buf, wd_buf, combine_ref,
                wg_sem, wu_sem, wd_sem):

    def copies(s, p):
        e = s // N_CHUNKS
        c = s % N_CHUNKS
        return (
            pltpu.make_async_copy(
                wgu_hbm.at[e, :, pl.ds(c * CHUNK, CHUNK)],
                wg_buf.at[p], wg_sem.at[p]),
            pltpu.make_async_copy(
                wgu_hbm.at[e, :, pl.ds(FFN + c * CHUNK, CHUNK)],
                wu_buf.at[p], wu_sem.at[p]),
            pltpu.make_async_copy(
                wd_hbm.at[e, pl.ds(c * CHUNK, CHUNK), :],
                wd_buf.at[p], wd_sem.at[p]),
        )

    def start(s, p):
        for cp in copies(s, p):
            cp.start()

    def wait(s, p):
        for cp in copies(s, p):
            cp.wait()

    x = x_ref[...]
    logits = jax.lax.dot_general(
        x, gw_ref[...], dimension_numbers=(((1,), (1,)), ((), ())),
        preferred_element_type=jnp.float32)
    logits_ref[...] = logits
    probs = jax.nn.softmax(logits, axis=-1)
    eidx = jax.lax.broadcasted_iota(jnp.int32, probs.shape, 1)
    p1 = jnp.max(probs, axis=-1, keepdims=True)
    i1 = jnp.min(jnp.where(probs >= p1, eidx, NUM_EXPERTS), axis=-1,
                 keepdims=True)
    sel1 = eidx == i1
    probs2 = jnp.where(sel1, -jnp.inf, probs)
    p2 = jnp.max(probs2, axis=-1, keepdims=True)
    i2 = jnp.min(jnp.where(probs2 >= p2, eidx, NUM_EXPERTS), axis=-1,
                 keepdims=True)
    sel2 = eidx == i2
    combine_ref[...] = (jnp.where(sel1, p1, 0.0)
                        + jnp.where(sel2, p2, 0.0)) / (p1 + p2)

    out_ref[...] = jnp.zeros(out_ref.shape, out_ref.dtype)

    for s in range(NBUF - 1):
        start(s, s)

    def group(g, carry):
        for p in range(NBUF):
            s = g * NBUF + p
            wait(s, p)

            @pl.when(s + NBUF - 1 < NSTEPS)
            def _():
                start(s + NBUF - 1, (p + NBUF - 1) % NBUF)

            gate = jnp.dot(x, wg_buf[p], preferred_element_type=jnp.float32)
            up = jnp.dot(x, wu_buf[p], preferred_element_type=jnp.float32)
            hidden = gate * jax.nn.sigmoid(gate) * up
            down = jnp.dot(hidden, wd_buf[p],
                           preferred_element_type=jnp.float32)
            combine = combine_ref[...]
            lane = jax.lax.broadcasted_iota(jnp.int32, combine.shape, 1)
            e = s // N_CHUNKS
            col = jnp.sum(jnp.where(lane == e, combine, 0.0), axis=-1,
                          keepdims=True)
            out_ref[...] = out_ref[...] + col * down
        return carry

    jax.lax.fori_loop(0, NSTEPS // NBUF, group, 0)


@functools.partial(jax.jit, static_argnames=())
def kernel(hidden_states, gate_w, w_gate_up, w_down):
    b, s, d = hidden_states.shape
    t = b * s
    x = hidden_states.reshape(t, d)

    out, logits = pl.pallas_call(
        _moe_kernel,
        in_specs=[
            pl.BlockSpec((t, d), lambda: (0, 0)),
            pl.BlockSpec((NUM_EXPERTS, d), lambda: (0, 0)),
            pl.BlockSpec(memory_space=pl.ANY),
            pl.BlockSpec(memory_space=pl.ANY),
        ],
        out_specs=[
            pl.BlockSpec((t, d), lambda: (0, 0)),
            pl.BlockSpec((t, NUM_EXPERTS), lambda: (0, 0)),
        ],
        out_shape=[
            jax.ShapeDtypeStruct((t, d), jnp.float32),
            jax.ShapeDtypeStruct((t, NUM_EXPERTS), jnp.float32),
        ],
        scratch_shapes=[
            pltpu.VMEM((NBUF, HIDDEN, CHUNK), jnp.float32),
            pltpu.VMEM((NBUF, HIDDEN, CHUNK), jnp.float32),
            pltpu.VMEM((NBUF, CHUNK, HIDDEN), jnp.float32),
            pltpu.VMEM((t, NUM_EXPERTS), jnp.float32),
            pltpu.SemaphoreType.DMA((NBUF,)),
            pltpu.SemaphoreType.DMA((NBUF,)),
            pltpu.SemaphoreType.DMA((NBUF,)),
        ],
    )(x, gate_w, w_gate_up, w_down)

    return out.reshape(b, s, d), logits


# speedup vs baseline: 1.0319x; 1.0213x over previous
import functools

import jax
import jax.numpy as jnp
from jax.experimental import pallas as pl
from jax.experimental.pallas import tpu as pltpu

HIDDEN = 1024
FFN = 2048
NUM_EXPERTS = 16
TOP_K = 2
CHUNK = 1024
N_CHUNKS = FFN // CHUNK


def _moe_kernel(x_ref, gw_ref, wg_ref, wu_ref, wd_ref, out_ref, logits_ref,
                combine_ref):
    e = pl.program_id(0)
    c = pl.program_id(1)
    first = jnp.logical_and(e == 0, c == 0)

    @pl.when(first)
    def _router():
        x = x_ref[...]
        logits = jax.lax.dot_general(
            x, gw_ref[...], dimension_numbers=(((1,), (1,)), ((), ())),
            preferred_element_type=jnp.float32)
        logits_ref[...] = logits
        probs = jax.nn.softmax(logits, axis=-1)
        eidx = jax.lax.broadcasted_iota(jnp.int32, probs.shape, 1)
        p1 = jnp.max(probs, axis=-1, keepdims=True)
        i1 = jnp.min(jnp.where(probs >= p1, eidx, NUM_EXPERTS), axis=-1,
                     keepdims=True)
        sel1 = eidx == i1
        probs2 = jnp.where(sel1, -jnp.inf, probs)
        p2 = jnp.max(probs2, axis=-1, keepdims=True)
        i2 = jnp.min(jnp.where(probs2 >= p2, eidx, NUM_EXPERTS), axis=-1,
                     keepdims=True)
        sel2 = eidx == i2
        denom = p1 + p2
        combine_ref[...] = (jnp.where(sel1, p1, 0.0)
                            + jnp.where(sel2, p2, 0.0)) / denom

    x = x_ref[...]
    gate = jnp.dot(x, wg_ref[0], preferred_element_type=jnp.float32)
    up = jnp.dot(x, wu_ref[0], preferred_element_type=jnp.float32)
    hidden = gate * jax.nn.sigmoid(gate) * up
    down = jnp.dot(hidden, wd_ref[0], preferred_element_type=jnp.float32)
    combine = combine_ref[...]
    lane = jax.lax.broadcasted_iota(jnp.int32, combine.shape, 1)
    col = jnp.sum(jnp.where(lane == e, combine, 0.0), axis=-1, keepdims=True)
    contrib = col * down

    @pl.when(first)
    def _init():
        out_ref[...] = contrib

    @pl.when(jnp.logical_not(first))
    def _acc():
        out_ref[...] = out_ref[...] + contrib


@functools.partial(jax.jit, static_argnames=())
def kernel(hidden_states, gate_w, w_gate_up, w_down):
    b, s, d = hidden_states.shape
    t = b * s
    x = hidden_states.reshape(t, d)

    out, logits = pl.pallas_call(
        _moe_kernel,
        grid=(NUM_EXPERTS, N_CHUNKS),
        in_specs=[
            pl.BlockSpec((t, d), lambda e, c: (0, 0)),
            pl.BlockSpec((NUM_EXPERTS, d), lambda e, c: (0, 0)),
            pl.BlockSpec((1, d, CHUNK), lambda e, c: (e, 0, c)),
            pl.BlockSpec((1, d, CHUNK), lambda e, c: (e, 0, N_CHUNKS + c)),
            pl.BlockSpec((1, CHUNK, d), lambda e, c: (e, c, 0)),
        ],
        out_specs=[
            pl.BlockSpec((t, d), lambda e, c: (0, 0)),
            pl.BlockSpec((t, NUM_EXPERTS), lambda e, c: (0, 0)),
        ],
        out_shape=[
            jax.ShapeDtypeStruct((t, d), jnp.float32),
            jax.ShapeDtypeStruct((t, NUM_EXPERTS), jnp.float32),
        ],
        scratch_shapes=[pltpu.VMEM((t, NUM_EXPERTS), jnp.float32)],
        compiler_params=pltpu.CompilerParams(
            dimension_semantics=("arbitrary", "arbitrary"),
        ),
    )(x, gate_w, w_gate_up, w_gate_up, w_down)

    return out.reshape(b, s, d), logits
